# Initial kernel scaffold; baseline (speedup 1.0000x reference)
import jax
import jax.numpy as jnp
from jax.experimental import pallas as pl


def kernel(node_feats, edge_attrs, edge_feats, lengths, edge_index, W_scalar, W_up, W1, W2, W3, W4, W_out, sn_weight, sn_bias, mean_weight, var_weight):
    raise NotImplementedError("write your pallas kernel here")


# speedup vs baseline: 2.1571x; 2.1571x over previous
import functools
import jax
import jax.numpy as jnp
from jax import lax
from jax.experimental import pallas as pl
from jax.experimental.pallas import tpu as pltpu
from jax.experimental.pallas import tpu_sc as plsc

R_MAX = 5.0
N_NODES = 10000
N_EDGES = 320000
D = 128
NUM_BESSEL = 8
AVG_NEIGH = 32.0
MLP_IN = 2 * D + 9

NUM_CORES = 2
NUM_SUBCORES = 16
NUM_WORKERS = NUM_CORES * NUM_SUBCORES
PER_TILE = N_EDGES // NUM_WORKERS
CHUNK = 80
N_ITERS = PER_TILE // CHUNK
N_PAD = 10240
ROWS_PER_TILE = N_PAD // NUM_SUBCORES

EDGE_BLOCK = 1600


def _node_tables_body(nf_ref, ws_ref, wu_ref, cat_ref, ns_ref):
    nf = nf_ref[...]
    ns = jnp.dot(nf, ws_ref[...], preferred_element_type=jnp.float32)
    nu = jnp.dot(nf, wu_ref[...], preferred_element_type=jnp.float32)
    cat_ref[:, :D] = ns
    cat_ref[:, D:] = nu
    ns_ref[...] = ns


def _node_tables(node_feats, ws, wu):
    nb = 1000
    grid = N_NODES // nb
    return pl.pallas_call(
        _node_tables_body,
        grid=(grid,),
        in_specs=[
            pl.BlockSpec((nb, D), lambda i: (i, 0)),
            pl.BlockSpec((D, D), lambda i: (0, 0)),
            pl.BlockSpec((D, D), lambda i: (0, 0)),
        ],
        out_specs=[
            pl.BlockSpec((nb, 2 * D), lambda i: (i, 0)),
            pl.BlockSpec((nb, D), lambda i: (i, 0)),
        ],
        out_shape=[
            jax.ShapeDtypeStruct((N_NODES, 2 * D), jnp.float32),
            jax.ShapeDtypeStruct((N_NODES, D), jnp.float32),
        ],
    )(node_feats, ws, wu)


def _sc_gather_body(send_hbm, recv_hbm, cat_hbm, ns_hbm, gs_hbm, gr_hbm,
                    idx_s, buf_s, idx_r, buf_r, sem_s, sem_r):
    wid = lax.axis_index("c") * NUM_SUBCORES + lax.axis_index("s")
    base = wid * PER_TILE

    def body(i, _):
        off = base + i * CHUNK
        pltpu.sync_copy(send_hbm.at[pl.ds(off, CHUNK)], idx_s)
        cp_s = pltpu.make_async_copy(cat_hbm.at[idx_s], buf_s, sem_s)
        cp_s.start()
        pltpu.sync_copy(recv_hbm.at[pl.ds(off, CHUNK)], idx_r)
        cp_r = pltpu.make_async_copy(ns_hbm.at[idx_r], buf_r, sem_r)
        cp_r.start()
        cp_s.wait()
        pltpu.sync_copy(buf_s, gs_hbm.at[pl.ds(off, CHUNK)])
        cp_r.wait()
        pltpu.sync_copy(buf_r, gr_hbm.at[pl.ds(off, CHUNK)])
        return 0

    lax.fori_loop(0, N_ITERS, body, 0)


def _sc_gather(sender, receiver, cat_tbl, ns_tbl):
    mesh = plsc.VectorSubcoreMesh(core_axis_name="c", subcore_axis_name="s")
    f = pl.kernel(
        _sc_gather_body,
        out_type=(
            jax.ShapeDtypeStruct((N_EDGES, 2 * D), jnp.float32),
            jax.ShapeDtypeStruct((N_EDGES, D), jnp.float32),
        ),
        mesh=mesh,
        scratch_types=[
            pltpu.VMEM((CHUNK,), jnp.int32),
            pltpu.VMEM((CHUNK, 2 * D), jnp.float32),
            pltpu.VMEM((CHUNK,), jnp.int32),
            pltpu.VMEM((CHUNK, D), jnp.float32),
            pltpu.SemaphoreType.DMA,
            pltpu.SemaphoreType.DMA,
        ],
    )
    return f(sender, receiver, cat_tbl, ns_tbl)


def _edge_mlp_body(params_ref, gs_ref, gr_ref, sca_ref,
                   w1ab_ref, c8_ref, drow_ref, ww1_ref, bw1_ref,
                   w2_ref, w3_ref, w4_ref, out_ref):
    mw0 = params_ref[0]
    vw0 = params_ref[1]

    el = sca_ref[:, 0:1]
    tt = sca_ref[:, 1:2]
    ll = sca_ref[:, 2:3]
    ea = sca_ref[:, 3:4]

    u = el * (1.0 / R_MAX)
    u2 = u * u
    u4 = u2 * u2
    u5 = u4 * u
    f = 1.0 - 21.0 * u5 + 35.0 * u5 * u - 15.0 * u5 * u2
    c = jnp.where(el < R_MAX, f, 0.0)

    n = lax.broadcasted_iota(jnp.int32, (1, NUM_BESSEL), 1).astype(
        jnp.float32) + 1.0
    npi_r = n * (jnp.pi / R_MAX)
    pref = jnp.sqrt(2.0 / R_MAX)
    bd = (pref * jnp.sin(npi_r * el) / el) * jnp.exp(-(npi_r * npi_r) * tt)

    gs = gs_ref[...]
    ns_s = gs[:, :D]
    nu_s = gs[:, D:]
    ns_r = gr_ref[...]

    s_lin = (jnp.sum(ns_s, axis=1, keepdims=True)
             + jnp.sum(ns_r, axis=1, keepdims=True)
             + jnp.sum(bd, axis=1, keepdims=True) + ll) * c
    s_sq = (jnp.sum(ns_s * ns_s, axis=1, keepdims=True)
            + jnp.sum(ns_r * ns_r, axis=1, keepdims=True)
            + jnp.sum(bd * bd, axis=1, keepdims=True) + ll * ll) * (c * c)
    mean_ln = s_lin * (1.0 / MLP_IN)
    var_ln = (s_sq - s_lin * mean_ln) * (1.0 / (MLP_IN - 1))
    inv_std = lax.rsqrt(vw0 * var_ln + 1e-5)

    x2 = jnp.concatenate([ns_s, ns_r], axis=1)
    p = jnp.dot(x2, w1ab_ref[...], preferred_element_type=jnp.float32)
    p += jnp.dot(bd, c8_ref[...], preferred_element_type=jnp.float32)
    p += ll * drow_ref[...]
    h = (c * p - (mw0 * mean_ln) * ww1_ref[...]) * inv_std + bw1_ref[...]
    h = h * jax.nn.sigmoid(h)
    h = jnp.dot(h, w2_ref[...], preferred_element_type=jnp.float32)
    h = h * jax.nn.sigmoid(h)
    h = jnp.dot(h, w3_ref[...], preferred_element_type=jnp.float32)
    h = h * jax.nn.sigmoid(h)
    tpw = jnp.dot(h, w4_ref[...], preferred_element_type=jnp.float32)
    out_ref[...] = nu_s * ea * tpw


def _edge_mlp(params, gs, gr, sca, w1ab, c8, drow, ww1, bw1, w2, w3, w4):
    grid = N_EDGES // EDGE_BLOCK
    wspec = lambda shape: pl.BlockSpec(shape, lambda i: (0, 0))
    return pl.pallas_call(
        _edge_mlp_body,
        grid=(grid,),
        in_specs=[
            pl.BlockSpec(memory_space=pltpu.SMEM),
            pl.BlockSpec((EDGE_BLOCK, 2 * D), lambda i: (i, 0)),
            pl.BlockSpec((EDGE_BLOCK, D), lambda i: (i, 0)),
            pl.BlockSpec((EDGE_BLOCK, 4), lambda i: (i, 0)),
            wspec((2 * D, 64)),
            wspec((NUM_BESSEL, 64)),
            wspec((1, 64)),
            wspec((1, 64)),
            wspec((1, 64)),
            wspec((64, 64)),
            wspec((64, 64)),
            wspec((64, D)),
        ],
        out_specs=pl.BlockSpec((EDGE_BLOCK, D), lambda i: (i, 0)),
        out_shape=jax.ShapeDtypeStruct((N_EDGES, D), jnp.float32),
        compiler_params=pltpu.CompilerParams(
            dimension_semantics=("arbitrary",)),
    )(params, gs, gr, sca, w1ab, c8, drow, ww1, bw1, w2, w3, w4)


def _sc_scatter_body(recv_hbm, mji_hbm, zeros_hbm, out_hbm,
                     idx_v, rows_v, acc_sh, sem):
    cid = lax.axis_index("c")
    sid = lax.axis_index("s")
    wid = cid * NUM_SUBCORES + sid
    rbase = sid * ROWS_PER_TILE

    pltpu.sync_copy(zeros_hbm, rows_v)

    def zbody(i, _):
        pltpu.sync_copy(rows_v, acc_sh.at[pl.ds(rbase + i * CHUNK, CHUNK)])
        return 0

    lax.fori_loop(0, ROWS_PER_TILE // CHUNK, zbody, 0)
    plsc.subcore_barrier()

    base = wid * PER_TILE

    def body(i, _):
        off = base + i * CHUNK
        pltpu.sync_copy(recv_hbm.at[pl.ds(off, CHUNK)], idx_v.at[0])
        pltpu.sync_copy(mji_hbm.at[pl.ds(off, CHUNK)], rows_v)
        pltpu.sync_copy(rows_v, acc_sh.at[idx_v.at[0]], add=True)
        return 0

    lax.fori_loop(0, N_ITERS, body, 0)
    plsc.subcore_barrier()

    n_valid = jnp.minimum(N_NODES - rbase, ROWS_PER_TILE)

    def wbody(i, _):
        r = rbase + i * CHUNK
        pltpu.sync_copy(acc_sh.at[pl.ds(r, CHUNK)], rows_v)
        pltpu.sync_copy(rows_v, out_hbm.at[cid, pl.ds(r, CHUNK)])
        return 0

    lax.fori_loop(0, n_valid // CHUNK, wbody, 0)


def _sc_scatter(receiver, mji, zeros):
    mesh = plsc.VectorSubcoreMesh(core_axis_name="c", subcore_axis_name="s")
    f = pl.kernel(
        _sc_scatter_body,
        out_type=jax.ShapeDtypeStruct((NUM_CORES, N_NODES, D), jnp.float32),
        mesh=mesh,
        scratch_types=[
            pltpu.VMEM((1, CHUNK), jnp.int32),
            pltpu.VMEM((CHUNK, D), jnp.float32),
            pltpu.VMEM_SHARED((N_PAD, D), jnp.float32),
            pltpu.SemaphoreType.DMA,
        ],
    )
    return f(receiver, mji, zeros)


def _out_linear_body(p0_ref, p1_ref, wo_ref, out_ref):
    m = p0_ref[...] + p1_ref[...]
    out_ref[...] = jnp.dot(m, wo_ref[...], preferred_element_type=jnp.float32)


def _out_linear(p0, p1, wo):
    nb = 1000
    grid = N_NODES // nb
    return pl.pallas_call(
        _out_linear_body,
        grid=(grid,),
        in_specs=[
            pl.BlockSpec((nb, D), lambda i: (i, 0)),
            pl.BlockSpec((nb, D), lambda i: (i, 0)),
            pl.BlockSpec((D, D), lambda i: (0, 0)),
        ],
        out_specs=pl.BlockSpec((nb, D), lambda i: (i, 0)),
        out_shape=jax.ShapeDtypeStruct((N_NODES, D), jnp.float32),
    )(p0, p1, wo)


def kernel(node_feats, edge_attrs, edge_feats, lengths, edge_index,
           W_scalar, W_up, W1, W2, W3, W4, W_out,
           sn_weight, sn_bias, mean_weight, var_weight):
    f32 = jnp.float32
    inv_sqrt_d = 1.0 / jnp.sqrt(f32(D))
    sender = edge_index[0].astype(jnp.int32)
    receiver = edge_index[1].astype(jnp.int32)

    ws = W_scalar * inv_sqrt_d
    wu = W_up * inv_sqrt_d
    w1w = (W1 * sn_weight[0][:, None]) * (1.0 / jnp.sqrt(f32(MLP_IN)))
    w1ab = w1w[: 2 * D]
    c8 = w1w[2 * D: 2 * D + NUM_BESSEL]
    drow = w1w[2 * D + NUM_BESSEL:]
    ww1 = (sn_weight[0] @ W1)[None, :] * (1.0 / jnp.sqrt(f32(MLP_IN)))
    bw1 = (sn_bias[0] @ W1)[None, :] * (1.0 / jnp.sqrt(f32(MLP_IN)))
    w2 = W2 * 0.125
    w3 = W3 * 0.125
    w4 = W4 * 0.125
    wo = W_out * (inv_sqrt_d / AVG_NEIGH)
    params = jnp.stack([jax.nn.softmax(mean_weight)[0],
                        jax.nn.softmax(var_weight)[0]])

    sca = jnp.concatenate(
        [edge_feats[0], edge_feats[1], lengths, edge_attrs], axis=1)

    cat_tbl, ns_tbl = _node_tables(node_feats, ws, wu)
    gs, gr = _sc_gather(sender, receiver, cat_tbl, ns_tbl)
    mji = _edge_mlp(params, gs, gr, sca, w1ab, c8, drow, ww1, bw1, w2, w3, w4)
    zeros = jnp.zeros((CHUNK, D), f32)
    partials = _sc_scatter(receiver, mji, zeros)
    message = _out_linear(partials[0], partials[1], wo)
    return message[:, :, None]


# speedup vs baseline: 2.3445x; 1.0869x over previous
import functools
import jax
import jax.numpy as jnp
from jax import lax
from jax.experimental import pallas as pl
from jax.experimental.pallas import tpu as pltpu
from jax.experimental.pallas import tpu_sc as plsc

R_MAX = 5.0
N_NODES = 10000
N_EDGES = 320000
D = 128
NUM_BESSEL = 8
AVG_NEIGH = 32.0
MLP_IN = 2 * D + 9

NUM_CORES = 2
NUM_SUBCORES = 16
NUM_WORKERS = NUM_CORES * NUM_SUBCORES
PER_TILE = N_EDGES // NUM_WORKERS
CHUNK = 80
N_ITERS = PER_TILE // CHUNK
N_PAD = 10240
ROWS_PER_TILE = N_PAD // NUM_SUBCORES

EDGE_BLOCK = 1600


def _node_tables_body(nf_ref, ws_ref, wu_ref, cat_ref, ns_ref):
    nf = nf_ref[...]
    ns = jnp.dot(nf, ws_ref[...], preferred_element_type=jnp.float32)
    nu = jnp.dot(nf, wu_ref[...], preferred_element_type=jnp.float32)
    cat_ref[:, :D] = ns
    cat_ref[:, D:] = nu
    ns_ref[...] = ns


def _node_tables(node_feats, ws, wu):
    nb = 1000
    grid = N_NODES // nb
    return pl.pallas_call(
        _node_tables_body,
        grid=(grid,),
        in_specs=[
            pl.BlockSpec((nb, D), lambda i: (i, 0)),
            pl.BlockSpec((D, D), lambda i: (0, 0)),
            pl.BlockSpec((D, D), lambda i: (0, 0)),
        ],
        out_specs=[
            pl.BlockSpec((nb, 2 * D), lambda i: (i, 0)),
            pl.BlockSpec((nb, D), lambda i: (i, 0)),
        ],
        out_shape=[
            jax.ShapeDtypeStruct((N_NODES, 2 * D), jnp.float32),
            jax.ShapeDtypeStruct((N_NODES, D), jnp.float32),
        ],
    )(node_feats, ws, wu)


def _sc_gather_body(send_hbm, recv_hbm, cat_hbm, ns_hbm, gs_hbm, gr_hbm,
                    idx_s, buf_s, idx_r, buf_r,
                    sem_gs0, sem_gs1, sem_gr0, sem_gr1,
                    sem_ws0, sem_ws1, sem_wr0, sem_wr1):
    wid = lax.axis_index("c") * NUM_SUBCORES + lax.axis_index("s")
    base = wid * PER_TILE
    sem_gs = (sem_gs0, sem_gs1)
    sem_gr = (sem_gr0, sem_gr1)
    sem_ws = (sem_ws0, sem_ws1)
    sem_wr = (sem_wr0, sem_wr1)

    def g_copies(c, b):
        off = base + c * CHUNK
        return (
            pltpu.make_async_copy(cat_hbm.at[idx_s.at[b]], buf_s.at[b],
                                  sem_gs[b]),
            pltpu.make_async_copy(ns_hbm.at[idx_r.at[b]], buf_r.at[b],
                                  sem_gr[b]),
        )

    def w_copies(c, b):
        off = base + c * CHUNK
        return (
            pltpu.make_async_copy(buf_s.at[b], gs_hbm.at[pl.ds(off, CHUNK)],
                                  sem_ws[b]),
            pltpu.make_async_copy(buf_r.at[b], gr_hbm.at[pl.ds(off, CHUNK)],
                                  sem_wr[b]),
        )

    def start(c, b):
        off = base + c * CHUNK
        pltpu.sync_copy(send_hbm.at[pl.ds(off, CHUNK)], idx_s.at[b])
        pltpu.sync_copy(recv_hbm.at[pl.ds(off, CHUNK)], idx_r.at[b])
        for cp in g_copies(c, b):
            cp.start()

    def finish(c, b):
        for cp in g_copies(c, b):
            cp.wait()
        for cp in w_copies(c, b):
            cp.start()

    def drain(c, b):
        for cp in w_copies(c, b):
            cp.wait()

    n_pairs = N_ITERS // 2
    start(0, 0)

    def body(k, _):
        start(2 * k + 1, 1)
        finish(2 * k, 0)
        drain(2 * k, 0)

        @pl.when(k < n_pairs - 1)
        def _():
            start(2 * k + 2, 0)

        finish(2 * k + 1, 1)
        drain(2 * k + 1, 1)
        return 0

    lax.fori_loop(0, n_pairs, body, 0)
    last = N_ITERS - 1
    start(last, 0)
    finish(last, 0)
    drain(last, 0)


def _sc_gather(sender, receiver, cat_tbl, ns_tbl):
    mesh = plsc.VectorSubcoreMesh(core_axis_name="c", subcore_axis_name="s")
    f = pl.kernel(
        _sc_gather_body,
        out_type=(
            jax.ShapeDtypeStruct((N_EDGES, 2 * D), jnp.float32),
            jax.ShapeDtypeStruct((N_EDGES, D), jnp.float32),
        ),
        mesh=mesh,
        scratch_types=[
            pltpu.VMEM((2, CHUNK), jnp.int32),
            pltpu.VMEM((2, CHUNK, 2 * D), jnp.float32),
            pltpu.VMEM((2, CHUNK), jnp.int32),
            pltpu.VMEM((2, CHUNK, D), jnp.float32),
        ] + [pltpu.SemaphoreType.DMA] * 8,
    )
    return f(sender, receiver, cat_tbl, ns_tbl)


def _edge_mlp_body(params_ref, gs_ref, gr_ref, sca_ref,
                   w1ab_ref, c8_ref, drow_ref, ww1_ref, bw1_ref,
                   w2_ref, w3_ref, w4_ref, out_ref):
    mw0 = params_ref[0]
    vw0 = params_ref[1]

    el = sca_ref[:, 0:1]
    tt = sca_ref[:, 1:2]
    ll = sca_ref[:, 2:3]
    ea = sca_ref[:, 3:4]

    u = el * (1.0 / R_MAX)
    u2 = u * u
    u4 = u2 * u2
    u5 = u4 * u
    f = 1.0 - 21.0 * u5 + 35.0 * u5 * u - 15.0 * u5 * u2
    c = jnp.where(el < R_MAX, f, 0.0)

    n = lax.broadcasted_iota(jnp.int32, (1, NUM_BESSEL), 1).astype(
        jnp.float32) + 1.0
    npi_r = n * (jnp.pi / R_MAX)
    pref = jnp.sqrt(2.0 / R_MAX)
    bd = (pref * jnp.sin(npi_r * el) / el) * jnp.exp(-(npi_r * npi_r) * tt)

    gs = gs_ref[...]
    ns_s = gs[:, :D]
    nu_s = gs[:, D:]
    ns_r = gr_ref[...]

    s_lin = (jnp.sum(ns_s, axis=1, keepdims=True)
             + jnp.sum(ns_r, axis=1, keepdims=True)
             + jnp.sum(bd, axis=1, keepdims=True) + ll) * c
    s_sq = (jnp.sum(ns_s * ns_s, axis=1, keepdims=True)
            + jnp.sum(ns_r * ns_r, axis=1, keepdims=True)
            + jnp.sum(bd * bd, axis=1, keepdims=True) + ll * ll) * (c * c)
    mean_ln = s_lin * (1.0 / MLP_IN)
    var_ln = (s_sq - s_lin * mean_ln) * (1.0 / (MLP_IN - 1))
    inv_std = lax.rsqrt(vw0 * var_ln + 1e-5)

    x2 = jnp.concatenate([ns_s, ns_r], axis=1)
    p = jnp.dot(x2, w1ab_ref[...], preferred_element_type=jnp.float32)
    p += jnp.dot(bd, c8_ref[...], preferred_element_type=jnp.float32)
    p += ll * drow_ref[...]
    h = (c * p - (mw0 * mean_ln) * ww1_ref[...]) * inv_std + bw1_ref[...]
    h = h * jax.nn.sigmoid(h)
    h = jnp.dot(h, w2_ref[...], preferred_element_type=jnp.float32)
    h = h * jax.nn.sigmoid(h)
    h = jnp.dot(h, w3_ref[...], preferred_element_type=jnp.float32)
    h = h * jax.nn.sigmoid(h)
    tpw = jnp.dot(h, w4_ref[...], preferred_element_type=jnp.float32)
    out_ref[...] = nu_s * ea * tpw


def _edge_mlp(params, gs, gr, sca, w1ab, c8, drow, ww1, bw1, w2, w3, w4):
    grid = N_EDGES // EDGE_BLOCK
    wspec = lambda shape: pl.BlockSpec(shape, lambda i: (0, 0))
    return pl.pallas_call(
        _edge_mlp_body,
        grid=(grid,),
        in_specs=[
            pl.BlockSpec(memory_space=pltpu.SMEM),
            pl.BlockSpec((EDGE_BLOCK, 2 * D), lambda i: (i, 0)),
            pl.BlockSpec((EDGE_BLOCK, D), lambda i: (i, 0)),
            pl.BlockSpec((EDGE_BLOCK, 4), lambda i: (i, 0)),
            wspec((2 * D, 64)),
            wspec((NUM_BESSEL, 64)),
            wspec((1, 64)),
            wspec((1, 64)),
            wspec((1, 64)),
            wspec((64, 64)),
            wspec((64, 64)),
            wspec((64, D)),
        ],
        out_specs=pl.BlockSpec((EDGE_BLOCK, D), lambda i: (i, 0)),
        out_shape=jax.ShapeDtypeStruct((N_EDGES, D), jnp.float32),
        compiler_params=pltpu.CompilerParams(
            dimension_semantics=("arbitrary",)),
    )(params, gs, gr, sca, w1ab, c8, drow, ww1, bw1, w2, w3, w4)


def _sc_scatter_body(recv_hbm, mji_hbm, zeros_hbm, out_hbm,
                     idx_v, rows_v, acc_sh, sem_l0, sem_l1):
    sem_l = (sem_l0, sem_l1)
    cid = lax.axis_index("c")
    sid = lax.axis_index("s")
    wid = cid * NUM_SUBCORES + sid
    rbase = sid * ROWS_PER_TILE

    pltpu.sync_copy(zeros_hbm, rows_v.at[0])

    def zbody(i, _):
        pltpu.sync_copy(rows_v.at[0],
                        acc_sh.at[pl.ds(rbase + i * CHUNK, CHUNK)])
        return 0

    lax.fori_loop(0, ROWS_PER_TILE // CHUNK, zbody, 0)
    plsc.subcore_barrier()

    base = wid * PER_TILE

    def load_copy(c, b):
        off = base + c * CHUNK
        return pltpu.make_async_copy(mji_hbm.at[pl.ds(off, CHUNK)],
                                     rows_v.at[b], sem_l[b])

    def start(c, b):
        off = base + c * CHUNK
        pltpu.sync_copy(recv_hbm.at[pl.ds(off, CHUNK)], idx_v.at[b])
        load_copy(c, b).start()

    def add(c, b):
        load_copy(c, b).wait()
        pltpu.sync_copy(rows_v.at[b], acc_sh.at[idx_v.at[b]], add=True)

    n_pairs = N_ITERS // 2
    start(0, 0)

    def body(k, _):
        start(2 * k + 1, 1)
        add(2 * k, 0)

        @pl.when(k < n_pairs - 1)
        def _():
            start(2 * k + 2, 0)

        add(2 * k + 1, 1)
        return 0

    lax.fori_loop(0, n_pairs, body, 0)
    last = N_ITERS - 1
    start(last, 0)
    add(last, 0)
    plsc.subcore_barrier()

    n_valid = jnp.minimum(N_NODES - rbase, ROWS_PER_TILE)

    def wbody(i, _):
        r = rbase + i * CHUNK
        pltpu.sync_copy(acc_sh.at[pl.ds(r, CHUNK)], rows_v.at[0])
        pltpu.sync_copy(rows_v.at[0], out_hbm.at[cid, pl.ds(r, CHUNK)])
        return 0

    lax.fori_loop(0, n_valid // CHUNK, wbody, 0)


def _sc_scatter(receiver, mji, zeros):
    mesh = plsc.VectorSubcoreMesh(core_axis_name="c", subcore_axis_name="s")
    f = pl.kernel(
        _sc_scatter_body,
        out_type=jax.ShapeDtypeStruct((NUM_CORES, N_NODES, D), jnp.float32),
        mesh=mesh,
        scratch_types=[
            pltpu.VMEM((2, CHUNK), jnp.int32),
            pltpu.VMEM((2, CHUNK, D), jnp.float32),
            pltpu.VMEM_SHARED((N_PAD, D), jnp.float32),
            pltpu.SemaphoreType.DMA,
            pltpu.SemaphoreType.DMA,
        ],
    )
    return f(receiver, mji, zeros)


def _out_linear_body(p0_ref, p1_ref, wo_ref, out_ref):
    m = p0_ref[...] + p1_ref[...]
    out_ref[...] = jnp.dot(m, wo_ref[...], preferred_element_type=jnp.float32)


def _out_linear(p0, p1, wo):
    nb = 1000
    grid = N_NODES // nb
    return pl.pallas_call(
        _out_linear_body,
        grid=(grid,),
        in_specs=[
            pl.BlockSpec((nb, D), lambda i: (i, 0)),
            pl.BlockSpec((nb, D), lambda i: (i, 0)),
            pl.BlockSpec((D, D), lambda i: (0, 0)),
        ],
        out_specs=pl.BlockSpec((nb, D), lambda i: (i, 0)),
        out_shape=jax.ShapeDtypeStruct((N_NODES, D), jnp.float32),
    )(p0, p1, wo)


def kernel(node_feats, edge_attrs, edge_feats, lengths, edge_index,
           W_scalar, W_up, W1, W2, W3, W4, W_out,
           sn_weight, sn_bias, mean_weight, var_weight):
    f32 = jnp.float32
    inv_sqrt_d = 1.0 / jnp.sqrt(f32(D))
    sender = edge_index[0].astype(jnp.int32)
    receiver = edge_index[1].astype(jnp.int32)

    ws = W_scalar * inv_sqrt_d
    wu = W_up * inv_sqrt_d
    w1w = (W1 * sn_weight[0][:, None]) * (1.0 / jnp.sqrt(f32(MLP_IN)))
    w1ab = w1w[: 2 * D]
    c8 = w1w[2 * D: 2 * D + NUM_BESSEL]
    drow = w1w[2 * D + NUM_BESSEL:]
    ww1 = (sn_weight[0] @ W1)[None, :] * (1.0 / jnp.sqrt(f32(MLP_IN)))
    bw1 = (sn_bias[0] @ W1)[None, :] * (1.0 / jnp.sqrt(f32(MLP_IN)))
    w2 = W2 * 0.125
    w3 = W3 * 0.125
    w4 = W4 * 0.125
    wo = W_out * (inv_sqrt_d / AVG_NEIGH)
    params = jnp.stack([jax.nn.softmax(mean_weight)[0],
                        jax.nn.softmax(var_weight)[0]])

    sca = jnp.concatenate(
        [edge_feats[0], edge_feats[1], lengths, edge_attrs], axis=1)

    cat_tbl, ns_tbl = _node_tables(node_feats, ws, wu)
    gs, gr = _sc_gather(sender, receiver, cat_tbl, ns_tbl)
    mji = _edge_mlp(params, gs, gr, sca, w1ab, c8, drow, ww1, bw1, w2, w3, w4)
    zeros = jnp.zeros((CHUNK, D), f32)
    partials = _sc_scatter(receiver, mji, zeros)
    message = _out_linear(partials[0], partials[1], wo)
    return message[:, :, None]


# speedup vs baseline: 2.5347x; 1.0811x over previous
import functools
import jax
import jax.numpy as jnp
from jax import lax
from jax.experimental import pallas as pl
from jax.experimental.pallas import tpu as pltpu
from jax.experimental.pallas import tpu_sc as plsc

R_MAX = 5.0
N_NODES = 10000
N_EDGES = 320000
D = 128
NUM_BESSEL = 8
AVG_NEIGH = 32.0
MLP_IN = 2 * D + 9

NUM_CORES = 2
NUM_SUBCORES = 16
NUM_WORKERS = NUM_CORES * NUM_SUBCORES
PER_TILE = N_EDGES // NUM_WORKERS
CHUNK = 80
N_ITERS = PER_TILE // CHUNK
N_PAD = 10240
ROWS_PER_TILE = N_PAD // NUM_SUBCORES

EDGE_BLOCK = 1600


def _node_tables_body(nf_ref, ws_ref, wu_ref, cat_ref, ns_ref):
    nf = nf_ref[...]
    ns = jnp.dot(nf, ws_ref[...], preferred_element_type=jnp.float32)
    nu = jnp.dot(nf, wu_ref[...], preferred_element_type=jnp.float32)
    cat_ref[:, :D] = ns
    cat_ref[:, D:] = nu
    ns_ref[...] = ns


def _node_tables(node_feats, ws, wu):
    nb = 1000
    grid = N_NODES // nb
    return pl.pallas_call(
        _node_tables_body,
        grid=(grid,),
        in_specs=[
            pl.BlockSpec((nb, D), lambda i: (i, 0)),
            pl.BlockSpec((D, D), lambda i: (0, 0)),
            pl.BlockSpec((D, D), lambda i: (0, 0)),
        ],
        out_specs=[
            pl.BlockSpec((nb, 2 * D), lambda i: (i, 0)),
            pl.BlockSpec((nb, D), lambda i: (i, 0)),
        ],
        out_shape=[
            jax.ShapeDtypeStruct((N_NODES, 2 * D), jnp.float32),
            jax.ShapeDtypeStruct((N_NODES, D), jnp.float32),
        ],
    )(node_feats, ws, wu)


def _sc_gather_body(send_hbm, recv_hbm, cat_hbm, ns_hbm, gs_hbm, gr_hbm,
                    idx_s, buf_s, idx_r, buf_r,
                    sem_gs0, sem_gs1, sem_gr0, sem_gr1,
                    sem_ws0, sem_ws1, sem_wr0, sem_wr1):
    wid = lax.axis_index("c") * NUM_SUBCORES + lax.axis_index("s")
    base = wid * PER_TILE
    sem_gs = (sem_gs0, sem_gs1)
    sem_gr = (sem_gr0, sem_gr1)
    sem_ws = (sem_ws0, sem_ws1)
    sem_wr = (sem_wr0, sem_wr1)

    def g_copies(c, b):
        off = base + c * CHUNK
        return (
            pltpu.make_async_copy(cat_hbm.at[idx_s.at[b]], buf_s.at[b],
                                  sem_gs[b]),
            pltpu.make_async_copy(ns_hbm.at[idx_r.at[b]], buf_r.at[b],
                                  sem_gr[b]),
        )

    def w_copies(c, b):
        off = base + c * CHUNK
        return (
            pltpu.make_async_copy(buf_s.at[b], gs_hbm.at[pl.ds(off, CHUNK)],
                                  sem_ws[b]),
            pltpu.make_async_copy(buf_r.at[b], gr_hbm.at[pl.ds(off, CHUNK)],
                                  sem_wr[b]),
        )

    def start(c, b):
        off = base + c * CHUNK
        pltpu.sync_copy(send_hbm.at[pl.ds(off, CHUNK)], idx_s.at[b])
        pltpu.sync_copy(recv_hbm.at[pl.ds(off, CHUNK)], idx_r.at[b])
        for cp in g_copies(c, b):
            cp.start()

    def finish(c, b):
        for cp in g_copies(c, b):
            cp.wait()
        for cp in w_copies(c, b):
            cp.start()

    def drain(c, b):
        for cp in w_copies(c, b):
            cp.wait()

    n_pairs = N_ITERS // 2
    start(0, 0)

    def body(k, _):
        start(2 * k + 1, 1)
        finish(2 * k, 0)
        drain(2 * k, 0)

        @pl.when(k < n_pairs - 1)
        def _():
            start(2 * k + 2, 0)

        finish(2 * k + 1, 1)
        drain(2 * k + 1, 1)
        return 0

    lax.fori_loop(0, n_pairs, body, 0)
    last = N_ITERS - 1
    start(last, 0)
    finish(last, 0)
    drain(last, 0)


def _sc_gather(sender, receiver, cat_tbl, ns_tbl):
    mesh = plsc.VectorSubcoreMesh(core_axis_name="c", subcore_axis_name="s")
    f = pl.kernel(
        _sc_gather_body,
        out_type=(
            jax.ShapeDtypeStruct((N_EDGES, 2 * D), jnp.float32),
            jax.ShapeDtypeStruct((N_EDGES, D), jnp.float32),
        ),
        mesh=mesh,
        scratch_types=[
            pltpu.VMEM((2, CHUNK), jnp.int32),
            pltpu.VMEM((2, CHUNK, 2 * D), jnp.float32),
            pltpu.VMEM((2, CHUNK), jnp.int32),
            pltpu.VMEM((2, CHUNK, D), jnp.float32),
        ] + [pltpu.SemaphoreType.DMA] * 8,
    )
    return f(sender, receiver, cat_tbl, ns_tbl)


def _edge_mlp_body(params_ref, gs_ref, gr_ref, sca_ref,
                   w1ab_ref, c8_ref, ones2d_ref, ones8_ref,
                   drow_ref, ww1_ref, bw1_ref,
                   w2_ref, w3_ref, w4_ref, out_ref):
    mw0 = params_ref[0]
    vw0 = params_ref[1]

    el = sca_ref[:, 0:1]
    tt = sca_ref[:, 1:2]
    ll = sca_ref[:, 2:3]
    ea = sca_ref[:, 3:4]

    u = el * (1.0 / R_MAX)
    u2 = u * u
    u4 = u2 * u2
    u5 = u4 * u
    f = 1.0 - 21.0 * u5 + 35.0 * u5 * u - 15.0 * u5 * u2
    c = jnp.where(el < R_MAX, f, 0.0)

    n = lax.broadcasted_iota(jnp.int32, (1, NUM_BESSEL), 1).astype(
        jnp.float32) + 1.0
    npi_r = n * (jnp.pi / R_MAX)
    pref = jnp.sqrt(2.0 / R_MAX)
    bd = (pref * jnp.sin(npi_r * el) / el) * jnp.exp(-(npi_r * npi_r) * tt)

    gs = gs_ref[...]
    nu_s = gs[:, D:]
    ns_r = gr_ref[...]

    x2 = jnp.concatenate([gs[:, :D], ns_r], axis=1)
    p_all = jnp.dot(x2, w1ab_ref[...], preferred_element_type=jnp.float32)
    bd_all = jnp.dot(bd, c8_ref[...], preferred_element_type=jnp.float32)
    sq_x2 = jnp.dot(x2 * x2, ones2d_ref[...],
                    preferred_element_type=jnp.float32)
    sq_bd = jnp.dot(bd * bd, ones8_ref[...],
                    preferred_element_type=jnp.float32)

    s_lin = (p_all[:, 64:65] + bd_all[:, 64:65] + ll) * c
    s_sq = (sq_x2 + sq_bd + ll * ll) * (c * c)
    mean_ln = s_lin * (1.0 / MLP_IN)
    var_ln = (s_sq - s_lin * mean_ln) * (1.0 / (MLP_IN - 1))
    inv_std = lax.rsqrt(vw0 * var_ln + 1e-5)

    p = p_all[:, :64] + bd_all[:, :64] + ll * drow_ref[...]
    h = (c * p - (mw0 * mean_ln) * ww1_ref[...]) * inv_std + bw1_ref[...]
    h = h * jax.nn.sigmoid(h)
    h = jnp.dot(h, w2_ref[...], preferred_element_type=jnp.float32)
    h = h * jax.nn.sigmoid(h)
    h = jnp.dot(h, w3_ref[...], preferred_element_type=jnp.float32)
    h = h * jax.nn.sigmoid(h)
    tpw = jnp.dot(h, w4_ref[...], preferred_element_type=jnp.float32)
    out_ref[...] = nu_s * ea * tpw


def _edge_mlp(params, gs, gr, sca, w1ab, c8, ones2d, ones8,
              drow, ww1, bw1, w2, w3, w4):
    grid = N_EDGES // EDGE_BLOCK
    wspec = lambda shape: pl.BlockSpec(shape, lambda i: (0, 0))
    return pl.pallas_call(
        _edge_mlp_body,
        grid=(grid,),
        in_specs=[
            pl.BlockSpec(memory_space=pltpu.SMEM),
            pl.BlockSpec((EDGE_BLOCK, 2 * D), lambda i: (i, 0)),
            pl.BlockSpec((EDGE_BLOCK, D), lambda i: (i, 0)),
            pl.BlockSpec((EDGE_BLOCK, 4), lambda i: (i, 0)),
            wspec((2 * D, 65)),
            wspec((NUM_BESSEL, 65)),
            wspec((2 * D, 1)),
            wspec((NUM_BESSEL, 1)),
            wspec((1, 64)),
            wspec((1, 64)),
            wspec((1, 64)),
            wspec((64, 64)),
            wspec((64, 64)),
            wspec((64, D)),
        ],
        out_specs=pl.BlockSpec((EDGE_BLOCK, D), lambda i: (i, 0)),
        out_shape=jax.ShapeDtypeStruct((N_EDGES, D), jnp.float32),
        compiler_params=pltpu.CompilerParams(
            dimension_semantics=("arbitrary",)),
    )(params, gs, gr, sca, w1ab, c8, ones2d, ones8,
      drow, ww1, bw1, w2, w3, w4)


def _sc_scatter_body(recv_hbm, mji_hbm, zeros_hbm, out_hbm,
                     idx_v, rows_v, acc_sh, sem_l0, sem_l1):
    sem_l = (sem_l0, sem_l1)
    cid = lax.axis_index("c")
    sid = lax.axis_index("s")
    wid = cid * NUM_SUBCORES + sid
    rbase = sid * ROWS_PER_TILE

    pltpu.sync_copy(zeros_hbm, rows_v.at[0])

    def zbody(i, _):
        pltpu.sync_copy(rows_v.at[0],
                        acc_sh.at[pl.ds(rbase + i * CHUNK, CHUNK)])
        return 0

    lax.fori_loop(0, ROWS_PER_TILE // CHUNK, zbody, 0)
    plsc.subcore_barrier()

    base = wid * PER_TILE

    def load_copy(c, b):
        off = base + c * CHUNK
        return pltpu.make_async_copy(mji_hbm.at[pl.ds(off, CHUNK)],
                                     rows_v.at[b], sem_l[b])

    def start(c, b):
        off = base + c * CHUNK
        pltpu.sync_copy(recv_hbm.at[pl.ds(off, CHUNK)], idx_v.at[b])
        load_copy(c, b).start()

    def add(c, b):
        load_copy(c, b).wait()
        pltpu.sync_copy(rows_v.at[b], acc_sh.at[idx_v.at[b]], add=True)

    n_pairs = N_ITERS // 2
    start(0, 0)

    def body(k, _):
        start(2 * k + 1, 1)
        add(2 * k, 0)

        @pl.when(k < n_pairs - 1)
        def _():
            start(2 * k + 2, 0)

        add(2 * k + 1, 1)
        return 0

    lax.fori_loop(0, n_pairs, body, 0)
    last = N_ITERS - 1
    start(last, 0)
    add(last, 0)
    plsc.subcore_barrier()

    n_valid = jnp.minimum(N_NODES - rbase, ROWS_PER_TILE)

    def wbody(i, _):
        r = rbase + i * CHUNK
        pltpu.sync_copy(acc_sh.at[pl.ds(r, CHUNK)], rows_v.at[0])
        pltpu.sync_copy(rows_v.at[0], out_hbm.at[cid, pl.ds(r, CHUNK)])
        return 0

    lax.fori_loop(0, n_valid // CHUNK, wbody, 0)


def _sc_scatter(receiver, mji, zeros):
    mesh = plsc.VectorSubcoreMesh(core_axis_name="c", subcore_axis_name="s")
    f = pl.kernel(
        _sc_scatter_body,
        out_type=jax.ShapeDtypeStruct((NUM_CORES, N_NODES, D), jnp.float32),
        mesh=mesh,
        scratch_types=[
            pltpu.VMEM((2, CHUNK), jnp.int32),
            pltpu.VMEM((2, CHUNK, D), jnp.float32),
            pltpu.VMEM_SHARED((N_PAD, D), jnp.float32),
            pltpu.SemaphoreType.DMA,
            pltpu.SemaphoreType.DMA,
        ],
    )
    return f(receiver, mji, zeros)


def _out_linear_body(p0_ref, p1_ref, wo_ref, out_ref):
    m = p0_ref[...] + p1_ref[...]
    out_ref[...] = jnp.dot(m, wo_ref[...], preferred_element_type=jnp.float32)


def _out_linear(p0, p1, wo):
    nb = 1000
    grid = N_NODES // nb
    return pl.pallas_call(
        _out_linear_body,
        grid=(grid,),
        in_specs=[
            pl.BlockSpec((nb, D), lambda i: (i, 0)),
            pl.BlockSpec((nb, D), lambda i: (i, 0)),
            pl.BlockSpec((D, D), lambda i: (0, 0)),
        ],
        out_specs=pl.BlockSpec((nb, D), lambda i: (i, 0)),
        out_shape=jax.ShapeDtypeStruct((N_NODES, D), jnp.float32),
    )(p0, p1, wo)


def kernel(node_feats, edge_attrs, edge_feats, lengths, edge_index,
           W_scalar, W_up, W1, W2, W3, W4, W_out,
           sn_weight, sn_bias, mean_weight, var_weight):
    f32 = jnp.float32
    inv_sqrt_d = 1.0 / jnp.sqrt(f32(D))
    sender = edge_index[0].astype(jnp.int32)
    receiver = edge_index[1].astype(jnp.int32)

    ws = W_scalar * inv_sqrt_d
    wu = W_up * inv_sqrt_d
    w1w = (W1 * sn_weight[0][:, None]) * (1.0 / jnp.sqrt(f32(MLP_IN)))
    ones_col = jnp.ones((2 * D, 1), f32)
    w1ab = jnp.concatenate([w1w[: 2 * D], ones_col], axis=1)
    c8 = jnp.concatenate(
        [w1w[2 * D: 2 * D + NUM_BESSEL], jnp.ones((NUM_BESSEL, 1), f32)],
        axis=1)
    ones8 = jnp.ones((NUM_BESSEL, 1), f32)
    drow = w1w[2 * D + NUM_BESSEL:]
    ww1 = (sn_weight[0] @ W1)[None, :] * (1.0 / jnp.sqrt(f32(MLP_IN)))
    bw1 = (sn_bias[0] @ W1)[None, :] * (1.0 / jnp.sqrt(f32(MLP_IN)))
    w2 = W2 * 0.125
    w3 = W3 * 0.125
    w4 = W4 * 0.125
    wo = W_out * (inv_sqrt_d / AVG_NEIGH)
    params = jnp.stack([jax.nn.softmax(mean_weight)[0],
                        jax.nn.softmax(var_weight)[0]])

    sca = jnp.concatenate(
        [edge_feats[0], edge_feats[1], lengths, edge_attrs], axis=1)

    cat_tbl, ns_tbl = _node_tables(node_feats, ws, wu)
    gs, gr = _sc_gather(sender, receiver, cat_tbl, ns_tbl)
    mji = _edge_mlp(params, gs, gr, sca, w1ab, c8, ones_col, ones8,
                    drow, ww1, bw1, w2, w3, w4)
    zeros = jnp.zeros((CHUNK, D), f32)
    partials = _sc_scatter(receiver, mji, zeros)
    message = _out_linear(partials[0], partials[1], wo)
    return message[:, :, None]


# speedup vs baseline: 3.4594x; 1.3648x over previous
import functools
import jax
import jax.numpy as jnp
from jax import lax
from jax.experimental import pallas as pl
from jax.experimental.pallas import tpu as pltpu
from jax.experimental.pallas import tpu_sc as plsc

R_MAX = 5.0
N_NODES = 10000
N_EDGES = 320000
D = 128
NUM_BESSEL = 8
AVG_NEIGH = 32.0
MLP_IN = 2 * D + 9

NUM_CORES = 2
NUM_SUBCORES = 16
NUM_WORKERS = NUM_CORES * NUM_SUBCORES
PER_TILE = N_EDGES // NUM_WORKERS
CHUNK = 80
N_ITERS = PER_TILE // CHUNK
N_PAD = 10240
ROWS_PER_TILE = N_PAD // NUM_SUBCORES

EDGE_BLOCK = 1280


def _node_tables_body(nf_ref, ws_ref, wu_ref, cat_ref, ns_ref):
    nf = nf_ref[...]
    ns = jnp.dot(nf, ws_ref[...], preferred_element_type=jnp.float32)
    nu = jnp.dot(nf, wu_ref[...], preferred_element_type=jnp.float32)
    cat_ref[:, :D] = ns
    cat_ref[:, D:] = nu
    ns_ref[...] = ns


def _node_tables(node_feats, ws, wu):
    nb = 1000
    grid = N_NODES // nb
    return pl.pallas_call(
        _node_tables_body,
        grid=(grid,),
        in_specs=[
            pl.BlockSpec((nb, D), lambda i: (i, 0)),
            pl.BlockSpec((D, D), lambda i: (0, 0)),
            pl.BlockSpec((D, D), lambda i: (0, 0)),
        ],
        out_specs=[
            pl.BlockSpec((nb, 2 * D), lambda i: (i, 0)),
            pl.BlockSpec((nb, D), lambda i: (i, 0)),
        ],
        out_shape=[
            jax.ShapeDtypeStruct((N_NODES, 2 * D), jnp.float32),
            jax.ShapeDtypeStruct((N_NODES, D), jnp.float32),
        ],
    )(node_feats, ws, wu)


def _sc_gather_body(send_hbm, recv_hbm, cat_hbm, ns_hbm, gs_hbm, gr_hbm,
                    idx_s, buf_s, idx_r, buf_r,
                    sem_gs0, sem_gs1, sem_gr0, sem_gr1,
                    sem_ws0, sem_ws1, sem_wr0, sem_wr1):
    wid = lax.axis_index("c") * NUM_SUBCORES + lax.axis_index("s")
    base = wid * PER_TILE
    sem_gs = (sem_gs0, sem_gs1)
    sem_gr = (sem_gr0, sem_gr1)
    sem_ws = (sem_ws0, sem_ws1)
    sem_wr = (sem_wr0, sem_wr1)

    def g_copies(c, b):
        off = base + c * CHUNK
        return (
            pltpu.make_async_copy(cat_hbm.at[idx_s.at[b]], buf_s.at[b],
                                  sem_gs[b]),
            pltpu.make_async_copy(ns_hbm.at[idx_r.at[b]], buf_r.at[b],
                                  sem_gr[b]),
        )

    def w_copies(c, b):
        off = base + c * CHUNK
        return (
            pltpu.make_async_copy(buf_s.at[b], gs_hbm.at[pl.ds(off, CHUNK)],
                                  sem_ws[b]),
            pltpu.make_async_copy(buf_r.at[b], gr_hbm.at[pl.ds(off, CHUNK)],
                                  sem_wr[b]),
        )

    def start(c, b):
        off = base + c * CHUNK
        pltpu.sync_copy(send_hbm.at[pl.ds(off, CHUNK)], idx_s.at[b])
        pltpu.sync_copy(recv_hbm.at[pl.ds(off, CHUNK)], idx_r.at[b])
        for cp in g_copies(c, b):
            cp.start()

    def finish(c, b):
        for cp in g_copies(c, b):
            cp.wait()
        for cp in w_copies(c, b):
            cp.start()

    def drain(c, b):
        for cp in w_copies(c, b):
            cp.wait()

    n_pairs = N_ITERS // 2
    start(0, 0)

    def body(k, _):
        start(2 * k + 1, 1)
        finish(2 * k, 0)
        drain(2 * k, 0)

        @pl.when(k < n_pairs - 1)
        def _():
            start(2 * k + 2, 0)

        finish(2 * k + 1, 1)
        drain(2 * k + 1, 1)
        return 0

    lax.fori_loop(0, n_pairs, body, 0)
    last = N_ITERS - 1
    start(last, 0)
    finish(last, 0)
    drain(last, 0)


def _sc_gather(sender, receiver, cat_tbl, ns_tbl):
    mesh = plsc.VectorSubcoreMesh(core_axis_name="c", subcore_axis_name="s")
    f = pl.kernel(
        _sc_gather_body,
        out_type=(
            jax.ShapeDtypeStruct((N_EDGES, 2 * D), jnp.float32),
            jax.ShapeDtypeStruct((N_EDGES, D), jnp.float32),
        ),
        mesh=mesh,
        scratch_types=[
            pltpu.VMEM((2, CHUNK), jnp.int32),
            pltpu.VMEM((2, CHUNK, 2 * D), jnp.float32),
            pltpu.VMEM((2, CHUNK), jnp.int32),
            pltpu.VMEM((2, CHUNK, D), jnp.float32),
        ] + [pltpu.SemaphoreType.DMA] * 8,
    )
    return f(sender, receiver, cat_tbl, ns_tbl)


def _edge_mlp_body(params_ref, gs_ref, gr_ref, sca_ref,
                   w1ab_ref, c8_ref, ones2d_ref, ones8_ref, eye8_ref,
                   drow_ref, ww1_ref, bw1_ref,
                   w2_ref, w3_ref, w4_ref, out_ref):
    mw0 = params_ref[0]
    vw0 = params_ref[1]

    el_l = sca_ref[0:1, :]
    tt_l = sca_ref[1:2, :]

    u = el_l * (1.0 / R_MAX)
    u2 = u * u
    u5 = u2 * u2 * u
    f = 1.0 - 21.0 * u5 + 35.0 * u5 * u - 15.0 * u5 * u2
    c_l = jnp.where(el_l < R_MAX, f, 0.0)

    n = lax.broadcasted_iota(jnp.int32, (NUM_BESSEL, 1), 0).astype(
        jnp.float32) + 1.0
    npi_r = n * (jnp.pi / R_MAX)
    pref = jnp.sqrt(2.0 / R_MAX)
    bd_l = (pref * jnp.sin(npi_r * el_l) / el_l) * jnp.exp(
        -(npi_r * npi_r) * tt_l)

    scl = jnp.concatenate([c_l, sca_ref[1:8, :]], axis=0)
    dn = (((0,), (0,)), ((), ()))
    cols = lax.dot_general(scl, eye8_ref[...], dn,
                           preferred_element_type=jnp.float32)
    c = cols[:, 0:1]
    ll = cols[:, 2:3]
    ea = cols[:, 3:4]

    gs = gs_ref[...]
    nu_s = gs[:, D:]
    ns_r = gr_ref[...]

    x2 = jnp.concatenate([gs[:, :D], ns_r], axis=1)
    p_all = jnp.dot(x2, w1ab_ref[...], preferred_element_type=jnp.float32)
    bd_all = lax.dot_general(bd_l, c8_ref[...], dn,
                             preferred_element_type=jnp.float32)
    sq_x2 = jnp.dot(x2 * x2, ones2d_ref[...],
                    preferred_element_type=jnp.float32)
    sq_bd = lax.dot_general(bd_l * bd_l, ones8_ref[...], dn,
                            preferred_element_type=jnp.float32)

    s_lin = (p_all[:, 64:65] + bd_all[:, 64:65] + ll) * c
    s_sq = (sq_x2 + sq_bd + ll * ll) * (c * c)
    mean_ln = s_lin * (1.0 / MLP_IN)
    var_ln = (s_sq - s_lin * mean_ln) * (1.0 / (MLP_IN - 1))
    inv_std = lax.rsqrt(vw0 * var_ln + 1e-5)

    p = p_all[:, :64] + bd_all[:, :64] + ll * drow_ref[...]
    h = (c * p - (mw0 * mean_ln) * ww1_ref[...]) * inv_std + bw1_ref[...]
    h = h * jax.nn.sigmoid(h)
    h = jnp.dot(h, w2_ref[...], preferred_element_type=jnp.float32)
    h = h * jax.nn.sigmoid(h)
    h = jnp.dot(h, w3_ref[...], preferred_element_type=jnp.float32)
    h = h * jax.nn.sigmoid(h)
    tpw = jnp.dot(h, w4_ref[...], preferred_element_type=jnp.float32)
    out_ref[...] = nu_s * ea * tpw


def _edge_mlp(params, gs, gr, sca_t, w1ab, c8, ones2d, ones8, eye8,
              drow, ww1, bw1, w2, w3, w4):
    grid = N_EDGES // EDGE_BLOCK
    wspec = lambda shape: pl.BlockSpec(shape, lambda i: (0, 0))
    return pl.pallas_call(
        _edge_mlp_body,
        grid=(grid,),
        in_specs=[
            pl.BlockSpec(memory_space=pltpu.SMEM),
            pl.BlockSpec((EDGE_BLOCK, 2 * D), lambda i: (i, 0)),
            pl.BlockSpec((EDGE_BLOCK, D), lambda i: (i, 0)),
            pl.BlockSpec((8, EDGE_BLOCK), lambda i: (0, i)),
            wspec((2 * D, 65)),
            wspec((NUM_BESSEL, 65)),
            wspec((2 * D, 1)),
            wspec((NUM_BESSEL, 1)),
            wspec((8, 8)),
            wspec((1, 64)),
            wspec((1, 64)),
            wspec((1, 64)),
            wspec((64, 64)),
            wspec((64, 64)),
            wspec((64, D)),
        ],
        out_specs=pl.BlockSpec((EDGE_BLOCK, D), lambda i: (i, 0)),
        out_shape=jax.ShapeDtypeStruct((N_EDGES, D), jnp.float32),
        compiler_params=pltpu.CompilerParams(
            dimension_semantics=("arbitrary",)),
    )(params, gs, gr, sca_t, w1ab, c8, ones2d, ones8, eye8,
      drow, ww1, bw1, w2, w3, w4)


def _sc_scatter_body(recv_hbm, mji_hbm, zeros_hbm, out_hbm,
                     idx_v, rows_v, acc_sh, sem_l0, sem_l1):
    sem_l = (sem_l0, sem_l1)
    cid = lax.axis_index("c")
    sid = lax.axis_index("s")
    wid = cid * NUM_SUBCORES + sid
    rbase = sid * ROWS_PER_TILE

    pltpu.sync_copy(zeros_hbm, rows_v.at[0])

    def zbody(i, _):
        pltpu.sync_copy(rows_v.at[0],
                        acc_sh.at[pl.ds(rbase + i * CHUNK, CHUNK)])
        return 0

    lax.fori_loop(0, ROWS_PER_TILE // CHUNK, zbody, 0)
    plsc.subcore_barrier()

    base = wid * PER_TILE

    def load_copy(c, b):
        off = base + c * CHUNK
        return pltpu.make_async_copy(mji_hbm.at[pl.ds(off, CHUNK)],
                                     rows_v.at[b], sem_l[b])

    def start(c, b):
        off = base + c * CHUNK
        pltpu.sync_copy(recv_hbm.at[pl.ds(off, CHUNK)], idx_v.at[b])
        load_copy(c, b).start()

    def add(c, b):
        load_copy(c, b).wait()
        pltpu.sync_copy(rows_v.at[b], acc_sh.at[idx_v.at[b]], add=True)

    n_pairs = N_ITERS // 2
    start(0, 0)

    def body(k, _):
        start(2 * k + 1, 1)
        add(2 * k, 0)

        @pl.when(k < n_pairs - 1)
        def _():
            start(2 * k + 2, 0)

        add(2 * k + 1, 1)
        return 0

    lax.fori_loop(0, n_pairs, body, 0)
    last = N_ITERS - 1
    start(last, 0)
    add(last, 0)
    plsc.subcore_barrier()

    n_valid = jnp.minimum(N_NODES - rbase, ROWS_PER_TILE)

    def wbody(i, _):
        r = rbase + i * CHUNK
        pltpu.sync_copy(acc_sh.at[pl.ds(r, CHUNK)], rows_v.at[0])
        pltpu.sync_copy(rows_v.at[0], out_hbm.at[cid, pl.ds(r, CHUNK)])
        return 0

    lax.fori_loop(0, n_valid // CHUNK, wbody, 0)


def _sc_scatter(receiver, mji, zeros):
    mesh = plsc.VectorSubcoreMesh(core_axis_name="c", subcore_axis_name="s")
    f = pl.kernel(
        _sc_scatter_body,
        out_type=jax.ShapeDtypeStruct((NUM_CORES, N_NODES, D), jnp.float32),
        mesh=mesh,
        scratch_types=[
            pltpu.VMEM((2, CHUNK), jnp.int32),
            pltpu.VMEM((2, CHUNK, D), jnp.float32),
            pltpu.VMEM_SHARED((N_PAD, D), jnp.float32),
            pltpu.SemaphoreType.DMA,
            pltpu.SemaphoreType.DMA,
        ],
    )
    return f(receiver, mji, zeros)


def _out_linear_body(p0_ref, p1_ref, wo_ref, out_ref):
    m = p0_ref[...] + p1_ref[...]
    out_ref[...] = jnp.dot(m, wo_ref[...], preferred_element_type=jnp.float32)


def _out_linear(p0, p1, wo):
    nb = 1000
    grid = N_NODES // nb
    return pl.pallas_call(
        _out_linear_body,
        grid=(grid,),
        in_specs=[
            pl.BlockSpec((nb, D), lambda i: (i, 0)),
            pl.BlockSpec((nb, D), lambda i: (i, 0)),
            pl.BlockSpec((D, D), lambda i: (0, 0)),
        ],
        out_specs=pl.BlockSpec((nb, D), lambda i: (i, 0)),
        out_shape=jax.ShapeDtypeStruct((N_NODES, D), jnp.float32),
    )(p0, p1, wo)


def kernel(node_feats, edge_attrs, edge_feats, lengths, edge_index,
           W_scalar, W_up, W1, W2, W3, W4, W_out,
           sn_weight, sn_bias, mean_weight, var_weight):
    f32 = jnp.float32
    inv_sqrt_d = 1.0 / jnp.sqrt(f32(D))
    sender = edge_index[0].astype(jnp.int32)
    receiver = edge_index[1].astype(jnp.int32)

    ws = W_scalar * inv_sqrt_d
    wu = W_up * inv_sqrt_d
    w1w = (W1 * sn_weight[0][:, None]) * (1.0 / jnp.sqrt(f32(MLP_IN)))
    ones_col = jnp.ones((2 * D, 1), f32)
    w1ab = jnp.concatenate([w1w[: 2 * D], ones_col], axis=1)
    c8 = jnp.concatenate(
        [w1w[2 * D: 2 * D + NUM_BESSEL], jnp.ones((NUM_BESSEL, 1), f32)],
        axis=1)
    ones8 = jnp.ones((NUM_BESSEL, 1), f32)
    drow = w1w[2 * D + NUM_BESSEL:]
    ww1 = (sn_weight[0] @ W1)[None, :] * (1.0 / jnp.sqrt(f32(MLP_IN)))
    bw1 = (sn_bias[0] @ W1)[None, :] * (1.0 / jnp.sqrt(f32(MLP_IN)))
    w2 = W2 * 0.125
    w3 = W3 * 0.125
    w4 = W4 * 0.125
    wo = W_out * (inv_sqrt_d / AVG_NEIGH)
    params = jnp.stack([jax.nn.softmax(mean_weight)[0],
                        jax.nn.softmax(var_weight)[0]])

    sca_t = jnp.concatenate(
        [edge_feats[0], edge_feats[1], lengths, edge_attrs,
         jnp.zeros((N_EDGES, 4), f32)], axis=1).T
    eye8 = jnp.eye(8, dtype=f32)

    cat_tbl, ns_tbl = _node_tables(node_feats, ws, wu)
    gs, gr = _sc_gather(sender, receiver, cat_tbl, ns_tbl)
    mji = _edge_mlp(params, gs, gr, sca_t, w1ab, c8, ones_col, ones8, eye8,
                    drow, ww1, bw1, w2, w3, w4)
    zeros = jnp.zeros((CHUNK, D), f32)
    partials = _sc_scatter(receiver, mji, zeros)
    message = _out_linear(partials[0], partials[1], wo)
    return message[:, :, None]
